# baseline (speedup 1.0000x reference)
import functools

import jax
import jax.numpy as jnp
from jax import lax
from jax.experimental import pallas as pl
from jax.experimental.pallas import tpu as pltpu
from jax.experimental.pallas import tpu_sc as plsc

N, C, H, W = 4, 96, 384, 384
HW = H * W
NPIX = N * HW
NW = 32
PPW = NPIX // NW
P = 128
NCHUNK = PPW // P
GRP = P // 16

_mesh = plsc.VectorSubcoreMesh(core_axis_name="c", subcore_axis_name="s")


def _f32(shape):
    return pltpu.VMEM(shape, jnp.float32)


def _i32(shape):
    return pltpu.VMEM(shape, jnp.int32)


@functools.partial(
    pl.kernel,
    out_type=jax.ShapeDtypeStruct((N, C, HW), jnp.float32),
    mesh=_mesh,
    scratch_types=[
        _f32((2 * P,)),
        [[_i32((P,)) for _ in range(4)] for _ in range(2)],
        [[_f32((P,)) for _ in range(4)] for _ in range(2)],
        [[_f32((P, C)) for _ in range(4)] for _ in range(2)],
        [_f32((C, P)) for _ in range(2)],
        [pltpu.SemaphoreType.DMA for _ in range(2)],
        [pltpu.SemaphoreType.DMA for _ in range(2)],
    ],
    compiler_params=pltpu.CompilerParams(
        use_tc_tiling_on_sc=False, needs_layout_passes=False),
)
def _grid_sample_sc(xt, m2, out, mv, idx, wgt, rows, outv, gsem, osem):
    cid = lax.axis_index("c")
    sid = lax.axis_index("s")
    wid = sid * 2 + cid
    base0 = wid * PPW
    n_img = wid // (NW // N)
    img_base = n_img * HW
    pbase0 = base0 - img_base
    iota = lax.iota(jnp.int32, 16)

    def stage(s, ci):
        base = base0 + ci * P
        pltpu.sync_copy(m2.at[pl.ds(2 * base, 2 * P)], mv)
        for g in range(GRP):
            sl = pl.ds(g * 16, 16)
            ev = (g * 16 + iota) * 2
            gx = plsc.load_gather(mv, [ev])
            gy = plsc.load_gather(mv, [ev + 1])
            ix = (gx + 1.0) * 0.5 * (W - 1)
            iy = (gy + 1.0) * 0.5 * (H - 1)
            ix0 = ix.astype(jnp.int32)
            iy0 = iy.astype(jnp.int32)
            wx1 = ix - ix0.astype(jnp.float32)
            wy1 = iy - iy0.astype(jnp.float32)
            wx0 = 1.0 - wx1
            wy0 = 1.0 - wy1
            ix1 = jnp.minimum(ix0 + 1, W - 1)
            iy1 = jnp.minimum(iy0 + 1, H - 1)
            row0 = iy0 * W + img_base
            row1 = iy1 * W + img_base
            idx[s][0][sl] = row0 + ix0
            idx[s][1][sl] = row0 + ix1
            idx[s][2][sl] = row1 + ix0
            idx[s][3][sl] = row1 + ix1
            wgt[s][0][sl] = wy0 * wx0
            wgt[s][1][sl] = wy0 * wx1
            wgt[s][2][sl] = wy1 * wx0
            wgt[s][3][sl] = wy1 * wx1
        for k in range(4):
            pltpu.async_copy(xt.at[idx[s][k]], rows[s][k], gsem[s])

    def wait_gathers(s):
        for k in range(4):
            pltpu.make_async_copy(xt.at[idx[s][k]], rows[s][k], gsem[s]).wait()

    def out_dst(ci):
        return out.at[n_img, :, pl.ds(pbase0 + ci * P, P)]

    def blend(s, ci):
        @pl.when(ci >= 2)
        def _():
            pltpu.make_async_copy(outv[s], out_dst(ci), osem[s]).wait()

        r0, r1, r2, r3 = rows[s]
        for g in range(GRP):
            gsl = pl.ds(g * 16, 16)
            prow = g * 16 + iota
            wa = wgt[s][0][gsl]
            wb = wgt[s][1][gsl]
            wc = wgt[s][2][gsl]
            wd = wgt[s][3][gsl]

            def cbody(c, carry):
                cols = jnp.full((16,), 0, jnp.int32) + c
                v0 = plsc.load_gather(r0, [prow, cols])
                v1 = plsc.load_gather(r1, [prow, cols])
                v2 = plsc.load_gather(r2, [prow, cols])
                v3 = plsc.load_gather(r3, [prow, cols])
                outv[s][c, gsl] = v0 * wa + v1 * wb + v2 * wc + v3 * wd
                return carry

            lax.fori_loop(0, C, cbody, 0, unroll=8)
        pltpu.async_copy(outv[s], out_dst(ci), osem[s])

    stage(0, 0)

    def body(cj, carry):
        ci0 = cj * 2
        ci1 = ci0 + 1
        stage(1, ci1)
        wait_gathers(0)
        blend(0, ci0)

        @pl.when(ci1 + 1 < NCHUNK)
        def _():
            stage(0, ci1 + 1)

        wait_gathers(1)
        blend(1, ci1)
        return carry

    lax.fori_loop(0, NCHUNK // 2, body, 0)
    pltpu.make_async_copy(outv[0], out_dst(NCHUNK - 2), osem[0]).wait()
    pltpu.make_async_copy(outv[1], out_dst(NCHUNK - 1), osem[1]).wait()


def kernel(x, m):
    xt = jnp.transpose(x, (0, 2, 3, 1)).reshape(NPIX, C)
    m2 = m.reshape(2 * NPIX)
    return _grid_sample_sc(xt, m2).reshape(N, C, H, W)


# speedup vs baseline: 2.6473x; 2.6473x over previous
import functools

import jax
import jax.numpy as jnp
from jax import lax
from jax.experimental import pallas as pl
from jax.experimental.pallas import tpu as pltpu
from jax.experimental.pallas import tpu_sc as plsc

N, C, H, W = 4, 96, 384, 384
HW = H * W
NPIX = N * HW
NW = 32
PPW = NPIX // NW
P = 128
NCHUNK = PPW // P
GRP = P // 16

_mesh = plsc.VectorSubcoreMesh(core_axis_name="c", subcore_axis_name="s")


def _f32(shape):
    return pltpu.VMEM(shape, jnp.float32)


def _i32(shape):
    return pltpu.VMEM(shape, jnp.int32)


@functools.partial(
    pl.kernel,
    out_type=jax.ShapeDtypeStruct((N, C, HW), jnp.float32),
    mesh=_mesh,
    scratch_types=[
        _f32((2 * P,)),
        [[_i32((P,)) for _ in range(4)] for _ in range(2)],
        [[_f32((P,)) for _ in range(4)] for _ in range(2)],
        [[_f32((P, C)) for _ in range(4)] for _ in range(2)],
        [_f32((C, P + 1)) for _ in range(2)],
        [pltpu.SemaphoreType.DMA for _ in range(2)],
        [pltpu.SemaphoreType.DMA for _ in range(2)],
    ],
    compiler_params=pltpu.CompilerParams(
        use_tc_tiling_on_sc=False, needs_layout_passes=False),
)
def _grid_sample_sc(xt, m2, out, mv, idx, wgt, rows, outv, gsem, osem):
    cid = lax.axis_index("c")
    sid = lax.axis_index("s")
    wid = sid * 2 + cid
    base0 = wid * PPW
    n_img = wid // (NW // N)
    img_base = n_img * HW
    pbase0 = base0 - img_base
    iota = lax.iota(jnp.int32, 16)

    def stage(s, ci):
        base = base0 + ci * P
        pltpu.sync_copy(m2.at[pl.ds(2 * base, 2 * P)], mv)
        for g in range(GRP):
            sl = pl.ds(g * 16, 16)
            ev = (g * 16 + iota) * 2
            gx = plsc.load_gather(mv, [ev])
            gy = plsc.load_gather(mv, [ev + 1])
            ix = (gx + 1.0) * 0.5 * (W - 1)
            iy = (gy + 1.0) * 0.5 * (H - 1)
            ix0 = ix.astype(jnp.int32)
            iy0 = iy.astype(jnp.int32)
            wx1 = ix - ix0.astype(jnp.float32)
            wy1 = iy - iy0.astype(jnp.float32)
            wx0 = 1.0 - wx1
            wy0 = 1.0 - wy1
            ix1 = jnp.minimum(ix0 + 1, W - 1)
            iy1 = jnp.minimum(iy0 + 1, H - 1)
            row0 = iy0 * W + img_base
            row1 = iy1 * W + img_base
            idx[s][0][sl] = row0 + ix0
            idx[s][1][sl] = row0 + ix1
            idx[s][2][sl] = row1 + ix0
            idx[s][3][sl] = row1 + ix1
            wgt[s][0][sl] = wy0 * wx0
            wgt[s][1][sl] = wy0 * wx1
            wgt[s][2][sl] = wy1 * wx0
            wgt[s][3][sl] = wy1 * wx1
        for k in range(4):
            pltpu.async_copy(xt.at[idx[s][k]], rows[s][k], gsem[s])

    def wait_gathers(s):
        for k in range(4):
            pltpu.make_async_copy(xt.at[idx[s][k]], rows[s][k], gsem[s]).wait()

    def out_dst(ci):
        return out.at[n_img, :, pl.ds(pbase0 + ci * P, P)]

    def blend(s, ci):
        @pl.when(ci >= 2)
        def _():
            pltpu.make_async_copy(
                outv[s].at[:, pl.ds(0, P)], out_dst(ci), osem[s]).wait()

        r0, r1, r2, r3 = rows[s]

        def gbody(g, carry):
            gsl = pl.ds(g * 16, 16)
            wa = wgt[s][0][gsl]
            wb = wgt[s][1][gsl]
            wc = wgt[s][2][gsl]
            wd = wgt[s][3][gsl]
            for l in range(16):
                p = g * 16 + l
                a = wa[l]
                b = wb[l]
                c = wc[l]
                d = wd[l]
                pvec = jnp.full((16,), 0, jnp.int32) + p
                for k in range(C // 16):
                    sl = pl.ds(k * 16, 16)
                    v = (r0[p, sl] * a + r1[p, sl] * b
                         + r2[p, sl] * c + r3[p, sl] * d)
                    plsc.store_scatter(outv[s], [k * 16 + iota, pvec], v)
            return carry

        lax.fori_loop(0, GRP, gbody, 0)
        pltpu.async_copy(outv[s].at[:, pl.ds(0, P)], out_dst(ci), osem[s])

    stage(0, 0)

    def body(cj, carry):
        ci0 = cj * 2
        ci1 = ci0 + 1
        stage(1, ci1)
        wait_gathers(0)
        blend(0, ci0)

        @pl.when(ci1 + 1 < NCHUNK)
        def _():
            stage(0, ci1 + 1)

        wait_gathers(1)
        blend(1, ci1)
        return carry

    lax.fori_loop(0, NCHUNK // 2, body, 0)
    pltpu.make_async_copy(
        outv[0].at[:, pl.ds(0, P)], out_dst(NCHUNK - 2), osem[0]).wait()
    pltpu.make_async_copy(
        outv[1].at[:, pl.ds(0, P)], out_dst(NCHUNK - 1), osem[1]).wait()


def kernel(x, m):
    xt = jnp.transpose(x, (0, 2, 3, 1)).reshape(NPIX, C)
    m2 = m.reshape(2 * NPIX)
    return _grid_sample_sc(xt, m2).reshape(N, C, H, W)
